# baseline (speedup 1.0000x reference)
import functools

import jax
import jax.numpy as jnp
from jax import lax
from jax.experimental import pallas as pl
from jax.experimental.pallas import tpu as pltpu
from jax.experimental.pallas import tpu_sc as plsc

N_NODES = 10000
N_EDGES = 320000
D_FEAT = 128
D_EDGE = 16
N_GRAPHS = 64

D_MSG = 32
N_ACC = 10240
E_PAD = 327680
NW = 32
EPT = E_PAD // NW
CH = 128
NCHUNK = EPT // CH
ZCOPY = N_ACC // (NW // 2) // CH

_HI = jax.lax.Precision.HIGHEST


def _node_mm_body(x_ref, w_ref, o_ref):
    o_ref[...] = jnp.dot(x_ref[...], w_ref[...])


def _node_mm(node_attr, w_top):
    blk = 1000
    return pl.pallas_call(
        _node_mm_body,
        grid=(N_NODES // blk,),
        in_specs=[
            pl.BlockSpec((blk, D_FEAT), lambda i: (i, 0)),
            pl.BlockSpec((D_FEAT, D_MSG), lambda i: (0, 0)),
        ],
        out_specs=pl.BlockSpec((blk, D_MSG), lambda i: (i, 0)),
        out_shape=jax.ShapeDtypeStruct((N_NODES, D_MSG), jnp.float32),
    )(node_attr, w_top)


def _edge_mm_body(x_ref, w_ref, b_ref, o_ref):
    o_ref[...] = jnp.dot(x_ref[...], w_ref[...]) + b_ref[...]


def _edge_mm(edge_attr8, w_big, b_big):
    blk = 2000
    rows = N_EDGES // 8
    return pl.pallas_call(
        _edge_mm_body,
        grid=(rows // blk,),
        in_specs=[
            pl.BlockSpec((blk, 128), lambda i: (i, 0)),
            pl.BlockSpec((128, 256), lambda i: (0, 0)),
            pl.BlockSpec((1, 256), lambda i: (0, 0)),
        ],
        out_specs=pl.BlockSpec((blk, 256), lambda i: (i, 0)),
        out_shape=jax.ShapeDtypeStruct((rows, 256), jnp.float32),
    )(edge_attr8, w_big, b_big)


N_REAL_CHUNKS = N_EDGES // CH
LASTN = N_REAL_CHUNKS - (NW - 1) * NCHUNK


def _sc_body(t_hbm, e_hbm, src2_hbm, dst2_hbm, out_hbm,
             sidx_all, didx_all, tb0, tb1, tb2, tb3, eb0, eb1, eb2, eb3,
             acc_sh, st0, st1, st2, st3, se0, se1, se2, se3):
    c = lax.axis_index("c")
    s = lax.axis_index("s")
    w = c * 16 + s
    base = w * NCHUNK
    nreal = jnp.minimum(NCHUNK, N_REAL_CHUNKS - base)

    @pl.when(w < NW - 1)
    def _():
        pltpu.sync_copy(src2_hbm.at[pl.ds(base, NCHUNK)], sidx_all)
        pltpu.sync_copy(dst2_hbm.at[pl.ds(base, NCHUNK)], didx_all)

    @pl.when(w == NW - 1)
    def _():
        pltpu.sync_copy(src2_hbm.at[pl.ds(base, LASTN)],
                        sidx_all.at[pl.ds(0, LASTN)])
        pltpu.sync_copy(dst2_hbm.at[pl.ds(base, LASTN)],
                        didx_all.at[pl.ds(0, LASTN)])

    @plsc.parallel_loop(0, CH, unroll=8)
    def _zrow(k):
        tb0[k, 0:16] = jnp.zeros((16,), jnp.float32)
        tb0[k, 16:32] = jnp.zeros((16,), jnp.float32)
    row0 = s * (N_ACC // 16)
    for z in range(ZCOPY):
        pltpu.sync_copy(tb0, acc_sh.at[pl.ds(row0 + z * CH, CH)])
    plsc.subcore_barrier()

    tb = (tb0, tb1, tb2, tb3)
    eb = (eb0, eb1, eb2, eb3)
    st = (st0, st1, st2, st3)
    se = (se0, se1, se2, se3)

    def issue(j, b):
        @pl.when(j < nreal)
        def _():
            pltpu.async_copy(t_hbm.at[sidx_all.at[j]], tb[b], st[b])
            pltpu.async_copy(e_hbm.at[pl.ds((base + j) * CH, CH)],
                             eb[b], se[b])

    def process(j, b):
        @pl.when(j < nreal)
        def _():
            pltpu.make_async_copy(t_hbm.at[sidx_all.at[j]], tb[b],
                                  st[b]).wait()
            pltpu.make_async_copy(e_hbm.at[pl.ds((base + j) * CH, CH)],
                                  eb[b], se[b]).wait()

            @plsc.parallel_loop(0, CH, unroll=8)
            def _row(k):
                a = tb[b][k, 0:16] + eb[b][k, 0:16]
                eb[b][k, 0:16] = jnp.maximum(a, 0.0)
                d = tb[b][k, 16:32] + eb[b][k, 16:32]
                eb[b][k, 16:32] = jnp.maximum(d, 0.0)

            pltpu.sync_copy(eb[b], acc_sh.at[didx_all.at[j]], add=True)

    issue(0, 0)
    issue(1, 1)

    def group(g, _):
        j0 = g * 4
        for b in range(4):
            process(j0 + b, b)
            issue(j0 + b + 2, (b + 2) % 4)
        return 0

    lax.fori_loop(0, NCHUNK // 4, group, 0)
    plsc.subcore_barrier()
    pltpu.sync_copy(acc_sh.at[pl.ds(row0, N_ACC // 16)],
                    out_hbm.at[c, pl.ds(row0, N_ACC // 16)])


def _sc_scatter(t, e, src2d, dst2d):
    mesh = plsc.VectorSubcoreMesh(core_axis_name="c", subcore_axis_name="s")
    fn = functools.partial(
        pl.kernel,
        mesh=mesh,
        out_type=jax.ShapeDtypeStruct((2, N_ACC, D_MSG), jnp.float32),
        scratch_types=(
            [pltpu.VMEM((NCHUNK, CH), jnp.int32)] * 2
            + [pltpu.VMEM((CH, D_MSG), jnp.float32)] * 8
            + [pltpu.VMEM_SHARED((N_ACC, D_MSG), jnp.float32)]
            + [pltpu.SemaphoreType.DMA] * 8
        ),
        compiler_params=pltpu.CompilerParams(use_tc_tiling_on_sc=False),
    )(_sc_body)
    return fn(t, e, src2d, dst2d)


def _tail_body(parts_ref, batch_ref, w1_ref, b1_ref, w2_ref, b2_ref,
               w3_ref, b3_ref, o_ref, accum_ref):
    i = pl.program_id(0)
    acc = parts_ref[0] + parts_ref[1]
    x1 = jnp.maximum(
        jnp.dot(acc, w1_ref[...]) + b1_ref[...], 0.0)
    bids = batch_ref[0, 0, :]
    oh = (bids[:, None]
          == lax.broadcasted_iota(jnp.int32, (bids.shape[0], N_GRAPHS), 1)
          ).astype(jnp.float32)
    part = lax.dot_general(oh, x1, (((0,), (0,)), ((), ())),
                           precision=_HI)

    @pl.when(i == 0)
    def _():
        accum_ref[...] = jnp.zeros_like(accum_ref)

    accum_ref[...] += part

    @pl.when(i == pl.num_programs(0) - 1)
    def _():
        g = jnp.maximum(
            jnp.dot(accum_ref[...], w2_ref[...])
            + b2_ref[...], 0.0)
        o_ref[...] = jnp.dot(g, w3_ref[...]) + b3_ref[...]


def _tail(parts, batch3d, w1p, b1p, w2p, b2p, w3p, b3p):
    blk = 1024
    nb = N_ACC // blk
    return pl.pallas_call(
        _tail_body,
        grid=(nb,),
        in_specs=[
            pl.BlockSpec((2, blk, D_MSG), lambda i: (0, i, 0)),
            pl.BlockSpec((1, 1, blk), lambda i: (i, 0, 0)),
            pl.BlockSpec((D_MSG, D_MSG), lambda i: (0, 0)),
            pl.BlockSpec((1, D_MSG), lambda i: (0, 0)),
            pl.BlockSpec((D_MSG, D_MSG), lambda i: (0, 0)),
            pl.BlockSpec((1, D_MSG), lambda i: (0, 0)),
            pl.BlockSpec((D_MSG, 128), lambda i: (0, 0)),
            pl.BlockSpec((1, 128), lambda i: (0, 0)),
        ],
        out_specs=pl.BlockSpec((N_GRAPHS, 128), lambda i: (0, 0)),
        out_shape=jax.ShapeDtypeStruct((N_GRAPHS, 128), jnp.float32),
        scratch_shapes=[pltpu.VMEM((N_GRAPHS, D_MSG), jnp.float32)],
    )(parts, batch3d, w1p, b1p, w2p, b2p, w3p, b3p)


def kernel(edge_index, node_attr, edge_attr, batch,
           W_mpl, b_mpl, W1, b1, W2, b2, W3, b3):
    src = edge_index[0].astype(jnp.int32)
    dst = edge_index[1].astype(jnp.int32)
    src2d = src.reshape(N_EDGES // CH, CH)
    dst2d = dst.reshape(N_EDGES // CH, CH)
    batch_pad = jnp.concatenate(
        [batch.astype(jnp.int32),
         jnp.full((N_ACC - N_NODES,), N_GRAPHS, jnp.int32)])
    batch3d = batch_pad.reshape(N_ACC // 1024, 1, 1024)

    w_top = jnp.zeros((D_FEAT, D_MSG), jnp.float32).at[:, :30].set(
        W_mpl[:D_FEAT])
    w_bot = jnp.zeros((D_EDGE, D_MSG), jnp.float32).at[:, :30].set(
        W_mpl[D_FEAT:])
    b_pad = jnp.zeros((1, D_MSG), jnp.float32).at[0, :30].set(b_mpl)
    w_big = jnp.zeros((128, 256), jnp.float32)
    for i in range(8):
        w_big = w_big.at[i * D_EDGE:(i + 1) * D_EDGE,
                         i * D_MSG:(i + 1) * D_MSG].set(w_bot)
    b_big = jnp.tile(b_pad, (1, 8))
    edge_attr8 = edge_attr.reshape(N_EDGES // 8, 128)
    w1p = jnp.zeros((D_MSG, D_MSG), jnp.float32).at[:30, :20].set(W1)
    b1p = jnp.zeros((1, D_MSG), jnp.float32).at[0, :20].set(b1)
    w2p = jnp.zeros((D_MSG, D_MSG), jnp.float32).at[:20, :10].set(W2)
    b2p = jnp.zeros((1, D_MSG), jnp.float32).at[0, :10].set(b2)
    w3p = jnp.zeros((D_MSG, 128), jnp.float32).at[:10, :1].set(W3)
    b3p = jnp.zeros((1, 128), jnp.float32).at[0, :1].set(b3)

    t = _node_mm(node_attr, w_top)
    e = _edge_mm(edge_attr8, w_big, b_big).reshape(N_EDGES, D_MSG)
    parts = _sc_scatter(t, e, src2d, dst2d)
    out = _tail(parts, batch3d, w1p, b1p, w2p, b2p, w3p, b3p)
    return out[:, :1]


# speedup vs baseline: 1.1056x; 1.1056x over previous
import functools

import jax
import jax.numpy as jnp
from jax import lax
from jax.experimental import pallas as pl
from jax.experimental.pallas import tpu as pltpu
from jax.experimental.pallas import tpu_sc as plsc

N_NODES = 10000
N_EDGES = 320000
D_FEAT = 128
D_EDGE = 16
N_GRAPHS = 64

D_MSG = 32
N_ACC = 10240
E_PAD = 327680
NW = 32
EPT = E_PAD // NW
CH = 128
NCHUNK = EPT // CH
ZCOPY = N_ACC // (NW // 2) // CH

_HI = jax.lax.Precision.HIGHEST


def _node_mm_body(x_ref, w_ref, o_ref):
    o_ref[...] = jnp.dot(x_ref[...], w_ref[...])


def _node_mm(node_attr, w_top):
    blk = 1000
    return pl.pallas_call(
        _node_mm_body,
        grid=(N_NODES // blk,),
        in_specs=[
            pl.BlockSpec((blk, D_FEAT), lambda i: (i, 0)),
            pl.BlockSpec((D_FEAT, D_MSG), lambda i: (0, 0)),
        ],
        out_specs=pl.BlockSpec((blk, D_MSG), lambda i: (i, 0)),
        out_shape=jax.ShapeDtypeStruct((N_NODES, D_MSG), jnp.float32),
    )(node_attr, w_top)


def _edge_mm_body(x_ref, w_ref, b_ref, o_ref):
    y = jnp.dot(x_ref[...], w_ref[...]) + b_ref[...]
    o_ref[...] = y.reshape(o_ref.shape)


def _edge_mm(edge_attr8, w_big, b_big):
    blk = 2000
    rows = N_EDGES // 8
    return pl.pallas_call(
        _edge_mm_body,
        grid=(rows // blk,),
        in_specs=[
            pl.BlockSpec((blk, 128), lambda i: (i, 0)),
            pl.BlockSpec((128, 256), lambda i: (0, 0)),
            pl.BlockSpec((1, 256), lambda i: (0, 0)),
        ],
        out_specs=pl.BlockSpec((2 * blk, 128), lambda i: (i, 0)),
        out_shape=jax.ShapeDtypeStruct((2 * rows, 128), jnp.float32),
    )(edge_attr8, w_big, b_big)


N_REAL_CHUNKS = N_EDGES // CH
LASTN = N_REAL_CHUNKS - (NW - 1) * NCHUNK


def _sc_body(t_hbm, e_hbm, ei3_hbm, out_hbm,
             sidx_all, didx_all, tb0, tb1, tb2, tb3, eb0, eb1, eb2, eb3,
             mbuf, acc_sh, st0, st1, st2, st3, se0, se1, se2, se3):
    c = lax.axis_index("c")
    s = lax.axis_index("s")
    w = c * 16 + s
    base = w * NCHUNK
    nreal = jnp.minimum(NCHUNK, N_REAL_CHUNKS - base)

    @pl.when(w < NW - 1)
    def _():
        pltpu.sync_copy(ei3_hbm.at[0, pl.ds(base, NCHUNK)], sidx_all)
        pltpu.sync_copy(ei3_hbm.at[1, pl.ds(base, NCHUNK)], didx_all)

    @pl.when(w == NW - 1)
    def _():
        pltpu.sync_copy(ei3_hbm.at[0, pl.ds(base, LASTN)],
                        sidx_all.at[pl.ds(0, LASTN)])
        pltpu.sync_copy(ei3_hbm.at[1, pl.ds(base, LASTN)],
                        didx_all.at[pl.ds(0, LASTN)])

    @plsc.parallel_loop(0, CH, unroll=8)
    def _zrow(k):
        tb0[k, 0:16] = jnp.zeros((16,), jnp.float32)
        tb0[k, 16:32] = jnp.zeros((16,), jnp.float32)
    row0 = s * (N_ACC // 16)
    for z in range(ZCOPY):
        pltpu.sync_copy(tb0, acc_sh.at[pl.ds(row0 + z * CH, CH)])
    plsc.subcore_barrier()

    tb = (tb0, tb1, tb2, tb3)
    eb = (eb0, eb1, eb2, eb3)
    st = (st0, st1, st2, st3)
    se = (se0, se1, se2, se3)

    def issue(j, b):
        @pl.when(j < nreal)
        def _():
            pltpu.async_copy(t_hbm.at[sidx_all.at[j]], tb[b], st[b])
            pltpu.async_copy(e_hbm.at[pl.ds((base + j) * (CH // 4), CH // 4)],
                             eb[b], se[b])

    def process(j, b):
        @pl.when(j < nreal)
        def _():
            pltpu.make_async_copy(t_hbm.at[sidx_all.at[j]], tb[b],
                                  st[b]).wait()
            pltpu.make_async_copy(
                e_hbm.at[pl.ds((base + j) * (CH // 4), CH // 4)],
                eb[b], se[b]).wait()

            @plsc.parallel_loop(0, CH // 4, unroll=4)
            def _row(r):
                for q in range(4):
                    k = r * 4 + q
                    col = q * 32
                    a = tb[b][k, 0:16] + eb[b][r, col:col + 16]
                    mbuf[k, 0:16] = jnp.maximum(a, 0.0)
                    d = tb[b][k, 16:32] + eb[b][r, col + 16:col + 32]
                    mbuf[k, 16:32] = jnp.maximum(d, 0.0)

            pltpu.sync_copy(mbuf, acc_sh.at[didx_all.at[j]], add=True)

    issue(0, 0)
    issue(1, 1)

    def group(g, _):
        j0 = g * 4
        for b in range(4):
            process(j0 + b, b)
            issue(j0 + b + 2, (b + 2) % 4)
        return 0

    lax.fori_loop(0, NCHUNK // 4, group, 0)
    plsc.subcore_barrier()
    pltpu.sync_copy(acc_sh.at[pl.ds(row0, N_ACC // 16)],
                    out_hbm.at[c, pl.ds(row0, N_ACC // 16)])


def _sc_scatter(t, e, ei3):
    mesh = plsc.VectorSubcoreMesh(core_axis_name="c", subcore_axis_name="s")
    fn = functools.partial(
        pl.kernel,
        mesh=mesh,
        out_type=jax.ShapeDtypeStruct((2, N_ACC, D_MSG), jnp.float32),
        scratch_types=(
            [pltpu.VMEM((NCHUNK, CH), jnp.int32)] * 2
            + [pltpu.VMEM((CH, D_MSG), jnp.float32)] * 4
            + [pltpu.VMEM((CH // 4, 128), jnp.float32)] * 4
            + [pltpu.VMEM((CH, D_MSG), jnp.float32)]
            + [pltpu.VMEM_SHARED((N_ACC, D_MSG), jnp.float32)]
            + [pltpu.SemaphoreType.DMA] * 8
        ),
        compiler_params=pltpu.CompilerParams(use_tc_tiling_on_sc=False),
    )(_sc_body)
    return fn(t, e, ei3)


def _tail_body(parts_ref, batch_ref, w1_ref, b1_ref, w2_ref, b2_ref,
               w3_ref, b3_ref, o_ref, accum_ref):
    i = pl.program_id(0)
    acc = parts_ref[0] + parts_ref[1]
    x1 = jnp.maximum(
        jnp.dot(acc, w1_ref[...]) + b1_ref[...], 0.0)
    bids = batch_ref[0, 0, :]
    oh = (bids[:, None]
          == lax.broadcasted_iota(jnp.int32, (bids.shape[0], N_GRAPHS), 1)
          ).astype(jnp.float32)
    part = lax.dot_general(oh, x1, (((0,), (0,)), ((), ())),
                           precision=_HI)

    @pl.when(i == 0)
    def _():
        accum_ref[...] = jnp.zeros_like(accum_ref)

    accum_ref[...] += part

    @pl.when(i == pl.num_programs(0) - 1)
    def _():
        g = jnp.maximum(
            jnp.dot(accum_ref[...], w2_ref[...])
            + b2_ref[...], 0.0)
        o_ref[...] = jnp.dot(g, w3_ref[...]) + b3_ref[...]


def _tail(parts, batch3d, w1p, b1p, w2p, b2p, w3p, b3p):
    blk = 1024
    nb = N_ACC // blk
    return pl.pallas_call(
        _tail_body,
        grid=(nb,),
        in_specs=[
            pl.BlockSpec((2, blk, D_MSG), lambda i: (0, i, 0)),
            pl.BlockSpec((1, 1, blk), lambda i: (i, 0, 0)),
            pl.BlockSpec((D_MSG, D_MSG), lambda i: (0, 0)),
            pl.BlockSpec((1, D_MSG), lambda i: (0, 0)),
            pl.BlockSpec((D_MSG, D_MSG), lambda i: (0, 0)),
            pl.BlockSpec((1, D_MSG), lambda i: (0, 0)),
            pl.BlockSpec((D_MSG, 128), lambda i: (0, 0)),
            pl.BlockSpec((1, 128), lambda i: (0, 0)),
        ],
        out_specs=pl.BlockSpec((N_GRAPHS, 128), lambda i: (0, 0)),
        out_shape=jax.ShapeDtypeStruct((N_GRAPHS, 128), jnp.float32),
        scratch_shapes=[pltpu.VMEM((N_GRAPHS, D_MSG), jnp.float32)],
    )(parts, batch3d, w1p, b1p, w2p, b2p, w3p, b3p)


def kernel(edge_index, node_attr, edge_attr, batch,
           W_mpl, b_mpl, W1, b1, W2, b2, W3, b3):
    ei3 = edge_index.astype(jnp.int32).reshape(2, N_EDGES // CH, CH)
    batch_pad = jnp.concatenate(
        [batch.astype(jnp.int32),
         jnp.full((N_ACC - N_NODES,), N_GRAPHS, jnp.int32)])
    batch3d = batch_pad.reshape(N_ACC // 1024, 1, 1024)

    w_top = jnp.zeros((D_FEAT, D_MSG), jnp.float32).at[:, :30].set(
        W_mpl[:D_FEAT])
    w_bot = jnp.zeros((D_EDGE, D_MSG), jnp.float32).at[:, :30].set(
        W_mpl[D_FEAT:])
    b_pad = jnp.zeros((1, D_MSG), jnp.float32).at[0, :30].set(b_mpl)
    w_big = jnp.zeros((128, 256), jnp.float32)
    for i in range(8):
        w_big = w_big.at[i * D_EDGE:(i + 1) * D_EDGE,
                         i * D_MSG:(i + 1) * D_MSG].set(w_bot)
    b_big = jnp.tile(b_pad, (1, 8))
    edge_attr8 = edge_attr.reshape(N_EDGES // 8, 128)
    w1p = jnp.zeros((D_MSG, D_MSG), jnp.float32).at[:30, :20].set(W1)
    b1p = jnp.zeros((1, D_MSG), jnp.float32).at[0, :20].set(b1)
    w2p = jnp.zeros((D_MSG, D_MSG), jnp.float32).at[:20, :10].set(W2)
    b2p = jnp.zeros((1, D_MSG), jnp.float32).at[0, :10].set(b2)
    w3p = jnp.zeros((D_MSG, 128), jnp.float32).at[:10, :1].set(W3)
    b3p = jnp.zeros((1, 128), jnp.float32).at[0, :1].set(b3)

    t = _node_mm(node_attr, w_top)
    e = _edge_mm(edge_attr8, w_big, b_big)
    parts = _sc_scatter(t, e, ei3)
    out = _tail(parts, batch3d, w1p, b1p, w2p, b2p, w3p, b3p)
    return out[:, :1]
